# baseline (speedup 1.0000x reference)
import functools

import jax
import jax.numpy as jnp
from jax.experimental import pallas as pl

_N = 5000
_K = 100
_NPAD = 5120
_R = 512
_C = 512
_SLOTS = 128
_HIGH = jax.lax.Precision.HIGHEST


def _supp_kernel(b_ref, bt_ref, o_ref):
    cb = pl.program_id(0)
    rb = pl.program_id(1)

    @pl.when(rb <= cb)
    def _compute():
        b = b_ref[...]
        bt = bt_ref[...]
        x1i, y1i, x2i, y2i = b[:, 0:1], b[:, 1:2], b[:, 2:3], b[:, 3:4]
        x1j, y1j, x2j, y2j = bt[0:1, :], bt[1:2, :], bt[2:3, :], bt[3:4, :]

        iw = jnp.minimum(x2i, x2j) - jnp.maximum(x1i, x1j)
        ih = jnp.minimum(y2i, y2j) - jnp.maximum(y1i, y1j)
        inter = jnp.maximum(iw, 0.0) * jnp.maximum(ih, 0.0)
        ai = (x2i - x1i) * (y2i - y1i)
        aj = (x2j - x1j) * (y2j - y1j)
        union = (ai + aj) - inter
        ovl = inter > 0.5 * union

        gi = rb * _R + jax.lax.broadcasted_iota(jnp.int32, (_R, 1), 0)
        gj = cb * _C + jax.lax.broadcasted_iota(jnp.int32, (1, _C), 1)
        supp = jnp.where(ovl & (gi < gj), 1.0, 0.0)
        col = jnp.max(supp, axis=0, keepdims=True)

        @pl.when(rb == 0)
        def _init():
            o_ref[...] = col

        @pl.when(rb != 0)
        def _acc():
            o_ref[...] = jnp.maximum(o_ref[...], col)


def _compact_kernel(supp_ref, data_ref, o_ref):
    supp = supp_ref[...]
    r_i = jax.lax.broadcasted_iota(jnp.int32, (40, 128), 0)
    l_i = jax.lax.broadcasted_iota(jnp.int32, (40, 128), 1)
    keep = (supp == 0.0) & ((r_i * 128 + l_i) < _N)
    kf = jnp.where(keep, 1.0, 0.0)

    u_r = jax.lax.broadcasted_iota(jnp.int32, (128, 128), 0)
    u_c = jax.lax.broadcasted_iota(jnp.int32, (128, 128), 1)
    upper = jnp.where(u_r <= u_c, 1.0, 0.0)
    incl = jnp.dot(kf, upper, precision=_HIGH)

    l_r = jax.lax.broadcasted_iota(jnp.int32, (40, 40), 0)
    l_c = jax.lax.broadcasted_iota(jnp.int32, (40, 40), 1)
    lower = jnp.where(l_r > l_c, 1.0, 0.0)
    offs = jnp.dot(lower, incl[:, 127:128], precision=_HIGH)

    rank = (incl + offs - kf).astype(jnp.int32)
    slot = jnp.where(keep, rank, jnp.int32(2**30))
    slot_flat = slot.reshape(1, _NPAD)
    p_i = jax.lax.broadcasted_iota(jnp.int32, (_SLOTS, 1), 0)
    onehot = jnp.where(p_i == slot_flat, 1.0, 0.0)
    o_ref[...] = jnp.dot(onehot, data_ref[...], precision=_HIGH)


@functools.partial(jax.jit, static_argnames=("interpret",))
def kernel(boxes, scores, interpret=False):
    neg, x1c, y1c, x2c, y2c = jax.lax.sort(
        (-scores, boxes[:, 0], boxes[:, 1], boxes[:, 2], boxes[:, 3]),
        num_keys=1, is_stable=True)
    s = -neg
    b = jnp.stack([x1c, y1c, x2c, y2c], axis=1)
    pad = _NPAD - _N
    b = jnp.pad(b, ((0, pad), (0, 0)))
    s = jnp.pad(s, (0, pad), constant_values=-1.0)
    bt = b.T

    nc, nr = _NPAD // _C, _NPAD // _R
    supp = pl.pallas_call(
        _supp_kernel,
        grid=(nc, nr),
        in_specs=[
            pl.BlockSpec((_R, 4), lambda cb, rb: (rb, 0)),
            pl.BlockSpec((4, _C), lambda cb, rb: (0, cb)),
        ],
        out_specs=pl.BlockSpec((1, _C), lambda cb, rb: (0, cb)),
        out_shape=jax.ShapeDtypeStruct((1, _NPAD), jnp.float32),
        interpret=interpret,
    )(b, bt)

    data = jnp.concatenate(
        [s[:, None], b, jnp.zeros((_NPAD, 3), jnp.float32)], axis=1)
    out8 = pl.pallas_call(
        _compact_kernel,
        out_shape=jax.ShapeDtypeStruct((_SLOTS, 8), jnp.float32),
        interpret=interpret,
    )(supp.reshape(_NPAD // 128, 128), data)

    return out8[:_K, :5]


# speedup vs baseline: 1.0098x; 1.0098x over previous
import functools

import jax
import jax.numpy as jnp
from jax.experimental import pallas as pl

_N = 5000
_K = 100
_NPAD = 5120
_R = 512
_C = 512
_SLOTS = 128
_HIGH = jax.lax.Precision.HIGHEST


def _supp_kernel(b_ref, bt_ref, o_ref):
    cb = pl.program_id(0)
    rb = pl.program_id(1)

    @pl.when(rb <= cb)
    def _compute():
        b = b_ref[...]
        bt = bt_ref[...]
        x1i, y1i, x2i, y2i = b[:, 0:1], b[:, 1:2], b[:, 2:3], b[:, 3:4]
        x1j, y1j, x2j, y2j = bt[0:1, :], bt[1:2, :], bt[2:3, :], bt[3:4, :]

        iw = jnp.minimum(x2i, x2j) - jnp.maximum(x1i, x1j)
        ih = jnp.minimum(y2i, y2j) - jnp.maximum(y1i, y1j)
        inter = jnp.maximum(iw, 0.0) * jnp.maximum(ih, 0.0)
        ai = (x2i - x1i) * (y2i - y1i)
        aj = (x2j - x1j) * (y2j - y1j)
        union = (ai + aj) - inter
        ovl = inter > 0.5 * union

        @pl.when(rb == cb)
        def _diag():
            gi = jax.lax.broadcasted_iota(jnp.int32, (_R, 1), 0)
            gj = jax.lax.broadcasted_iota(jnp.int32, (1, _C), 1)
            col = jnp.max(jnp.where(ovl & (gi < gj), 1.0, 0.0),
                          axis=0, keepdims=True)
            @pl.when(rb == 0)
            def _init():
                o_ref[...] = col
            @pl.when(rb != 0)
            def _acc():
                o_ref[...] = jnp.maximum(o_ref[...], col)

        @pl.when(rb < cb)
        def _off():
            col = jnp.max(jnp.where(ovl, 1.0, 0.0), axis=0, keepdims=True)
            @pl.when(rb == 0)
            def _init():
                o_ref[...] = col
            @pl.when(rb != 0)
            def _acc():
                o_ref[...] = jnp.maximum(o_ref[...], col)


def _compact_kernel(supp_ref, data_ref, o_ref):
    supp = supp_ref[...]
    r_i = jax.lax.broadcasted_iota(jnp.int32, (40, 128), 0)
    l_i = jax.lax.broadcasted_iota(jnp.int32, (40, 128), 1)
    keep = (supp == 0.0) & ((r_i * 128 + l_i) < _N)
    kf = jnp.where(keep, 1.0, 0.0)

    u_r = jax.lax.broadcasted_iota(jnp.int32, (128, 128), 0)
    u_c = jax.lax.broadcasted_iota(jnp.int32, (128, 128), 1)
    upper = jnp.where(u_r <= u_c, 1.0, 0.0)
    incl = jnp.dot(kf, upper, precision=_HIGH)

    l_r = jax.lax.broadcasted_iota(jnp.int32, (40, 40), 0)
    l_c = jax.lax.broadcasted_iota(jnp.int32, (40, 40), 1)
    lower = jnp.where(l_r > l_c, 1.0, 0.0)
    offs = jnp.dot(lower, incl[:, 127:128], precision=_HIGH)

    rank = (incl + offs - kf).astype(jnp.int32)
    slot = jnp.where(keep, rank, jnp.int32(2**30))
    slot_flat = slot.reshape(1, _NPAD)
    p_i = jax.lax.broadcasted_iota(jnp.int32, (_SLOTS, 1), 0)
    onehot = jnp.where(p_i == slot_flat, 1.0, 0.0)
    o_ref[...] = jnp.dot(onehot, data_ref[...], precision=_HIGH)


@functools.partial(jax.jit, static_argnames=("interpret",))
def kernel(boxes, scores, interpret=False):
    neg, x1c, y1c, x2c, y2c = jax.lax.sort(
        (-scores, boxes[:, 0], boxes[:, 1], boxes[:, 2], boxes[:, 3]),
        num_keys=1, is_stable=True)
    s = -neg
    b = jnp.stack([x1c, y1c, x2c, y2c], axis=1)
    pad = _NPAD - _N
    b = jnp.pad(b, ((0, pad), (0, 0)))
    s = jnp.pad(s, (0, pad), constant_values=-1.0)
    bt = b.T

    nc, nr = _NPAD // _C, _NPAD // _R
    supp = pl.pallas_call(
        _supp_kernel,
        grid=(nc, nr),
        in_specs=[
            pl.BlockSpec((_R, 4), lambda cb, rb: (rb, 0)),
            pl.BlockSpec((4, _C), lambda cb, rb: (0, cb)),
        ],
        out_specs=pl.BlockSpec((1, _C), lambda cb, rb: (0, cb)),
        out_shape=jax.ShapeDtypeStruct((1, _NPAD), jnp.float32),
        interpret=interpret,
    )(b, bt)

    data = jnp.concatenate(
        [s[:, None], b, jnp.zeros((_NPAD, 3), jnp.float32)], axis=1)
    out8 = pl.pallas_call(
        _compact_kernel,
        out_shape=jax.ShapeDtypeStruct((_SLOTS, 8), jnp.float32),
        interpret=interpret,
    )(supp.reshape(_NPAD // 128, 128), data)

    return out8[:_K, :5]


# speedup vs baseline: 1.3253x; 1.3124x over previous
import functools

import jax
import jax.numpy as jnp
import numpy as np
from jax.experimental import pallas as pl
from jax.experimental.pallas import tpu as pltpu

_N = 5000
_K = 100
_NPAD = 5120
_R = 512
_C = 512
_SLOTS = 128
_HIGH = jax.lax.Precision.HIGHEST

_NB = _NPAD // _R
_STEPS = [(cb, rb) for cb in range(_NB) for rb in range(cb + 1)]
_TABLE = np.asarray(_STEPS, dtype=np.int32).T.copy()


def _supp_kernel(tb_ref, b_ref, bt_ref, o_ref):
    i = pl.program_id(0)
    cb = tb_ref[0, i]
    rb = tb_ref[1, i]

    b = b_ref[...]
    bt = bt_ref[...]
    x1i, y1i, x2i, y2i = b[:, 0:1], b[:, 1:2], b[:, 2:3], b[:, 3:4]
    x1j, y1j, x2j, y2j = bt[0:1, :], bt[1:2, :], bt[2:3, :], bt[3:4, :]

    iw = jnp.minimum(x2i, x2j) - jnp.maximum(x1i, x1j)
    ih = jnp.minimum(y2i, y2j) - jnp.maximum(y1i, y1j)
    inter = jnp.maximum(iw, 0.0) * jnp.maximum(ih, 0.0)
    ai = (x2i - x1i) * (y2i - y1i)
    aj = (x2j - x1j) * (y2j - y1j)
    union = (ai + aj) - inter
    ovl = inter > 0.5 * union

    @pl.when(rb == cb)
    def _diag():
        gi = jax.lax.broadcasted_iota(jnp.int32, (_R, 1), 0)
        gj = jax.lax.broadcasted_iota(jnp.int32, (1, _C), 1)
        col = jnp.where(jnp.any(ovl & (gi < gj), axis=0, keepdims=True),
                        1.0, 0.0)
        @pl.when(rb == 0)
        def _init():
            o_ref[...] = col
        @pl.when(rb != 0)
        def _acc():
            o_ref[...] = jnp.maximum(o_ref[...], col)

    @pl.when(rb < cb)
    def _off():
        col = jnp.where(jnp.any(ovl, axis=0, keepdims=True), 1.0, 0.0)
        @pl.when(rb == 0)
        def _init():
            o_ref[...] = col
        @pl.when(rb != 0)
        def _acc():
            o_ref[...] = jnp.maximum(o_ref[...], col)


def _compact_kernel(supp_ref, data_ref, o_ref):
    supp = supp_ref[...]
    r_i = jax.lax.broadcasted_iota(jnp.int32, (40, 128), 0)
    l_i = jax.lax.broadcasted_iota(jnp.int32, (40, 128), 1)
    keep = (supp == 0.0) & ((r_i * 128 + l_i) < _N)
    kf = jnp.where(keep, 1.0, 0.0)

    u_r = jax.lax.broadcasted_iota(jnp.int32, (128, 128), 0)
    u_c = jax.lax.broadcasted_iota(jnp.int32, (128, 128), 1)
    upper = jnp.where(u_r <= u_c, 1.0, 0.0)
    incl = jnp.dot(kf, upper, precision=_HIGH)

    l_r = jax.lax.broadcasted_iota(jnp.int32, (40, 40), 0)
    l_c = jax.lax.broadcasted_iota(jnp.int32, (40, 40), 1)
    lower = jnp.where(l_r > l_c, 1.0, 0.0)
    offs = jnp.dot(lower, incl[:, 127:128], precision=_HIGH)

    rank = (incl + offs - kf).astype(jnp.int32)
    slot = jnp.where(keep, rank, jnp.int32(2**30))
    slot_flat = slot.reshape(1, _NPAD)
    p_i = jax.lax.broadcasted_iota(jnp.int32, (_SLOTS, 1), 0)
    onehot = jnp.where(p_i == slot_flat, 1.0, 0.0)
    o_ref[...] = jnp.dot(onehot, data_ref[...], precision=_HIGH)


@functools.partial(jax.jit, static_argnames=("interpret",))
def kernel(boxes, scores, interpret=False):
    neg, x1c, y1c, x2c, y2c = jax.lax.sort(
        (-scores, boxes[:, 0], boxes[:, 1], boxes[:, 2], boxes[:, 3]),
        num_keys=1, is_stable=True)
    s = -neg
    b = jnp.stack([x1c, y1c, x2c, y2c], axis=1)
    pad = _NPAD - _N
    b = jnp.pad(b, ((0, pad), (0, 0)))
    s = jnp.pad(s, (0, pad), constant_values=-1.0)
    bt = b.T

    supp = pl.pallas_call(
        _supp_kernel,
        grid_spec=pltpu.PrefetchScalarGridSpec(
            num_scalar_prefetch=1,
            grid=(len(_STEPS),),
            in_specs=[
                pl.BlockSpec((_R, 4), lambda i, tb: (tb[1, i], 0)),
                pl.BlockSpec((4, _C), lambda i, tb: (0, tb[0, i])),
            ],
            out_specs=pl.BlockSpec((1, _C), lambda i, tb: (0, tb[0, i])),
        ),
        out_shape=jax.ShapeDtypeStruct((1, _NPAD), jnp.float32),
        interpret=interpret,
    )(jnp.asarray(_TABLE), b, bt)

    data = jnp.concatenate(
        [s[:, None], b, jnp.zeros((_NPAD, 3), jnp.float32)], axis=1)
    out8 = pl.pallas_call(
        _compact_kernel,
        out_shape=jax.ShapeDtypeStruct((_SLOTS, 8), jnp.float32),
        interpret=interpret,
    )(supp.reshape(_NPAD // 128, 128), data)

    return out8[:_K, :5]


# speedup vs baseline: 1.3300x; 1.0036x over previous
import functools

import jax
import jax.numpy as jnp
import numpy as np
from jax.experimental import pallas as pl
from jax.experimental.pallas import tpu as pltpu

_N = 5000
_K = 100
_NPAD = 5120
_R = 512
_C = 512
_SLOTS = 128
_HIGH = jax.lax.Precision.HIGHEST

_NB = _NPAD // _R
_STEPS = [(cb, rb) for cb in range(_NB) for rb in range(cb + 1)]
_TABLE = np.asarray(_STEPS, dtype=np.int32).T.copy()


def _supp_kernel(tb_ref, b_ref, bt_ref, o_ref):
    i = pl.program_id(0)
    cb = tb_ref[0, i]
    rb = tb_ref[1, i]

    b = b_ref[...]
    bt = bt_ref[...]
    x1i, y1i, x2i, y2i = b[:, 0:1], b[:, 1:2], b[:, 2:3], b[:, 3:4]
    x1j, y1j, x2j, y2j = bt[0:1, :], bt[1:2, :], bt[2:3, :], bt[3:4, :]

    iw = jnp.minimum(x2i, x2j) - jnp.maximum(x1i, x1j)
    ih = jnp.minimum(y2i, y2j) - jnp.maximum(y1i, y1j)
    inter = jnp.maximum(iw, 0.0) * jnp.maximum(ih, 0.0)
    ai = (x2i - x1i) * (y2i - y1i)
    aj = (x2j - x1j) * (y2j - y1j)
    union = (ai + aj) - inter
    ovl = inter > 0.5 * union

    @pl.when(rb == cb)
    def _diag():
        gi = jax.lax.broadcasted_iota(jnp.int32, (_R, 1), 0)
        gj = jax.lax.broadcasted_iota(jnp.int32, (1, _C), 1)
        col = jnp.where(jnp.any(ovl & (gi < gj), axis=0, keepdims=True),
                        1.0, 0.0).reshape(1, _C // 128, 128)
        @pl.when(rb == 0)
        def _init():
            o_ref[...] = col
        @pl.when(rb != 0)
        def _acc():
            o_ref[...] = jnp.maximum(o_ref[...], col)

    @pl.when(rb < cb)
    def _off():
        col = jnp.where(jnp.any(ovl, axis=0, keepdims=True),
                        1.0, 0.0).reshape(1, _C // 128, 128)
        @pl.when(rb == 0)
        def _init():
            o_ref[...] = col
        @pl.when(rb != 0)
        def _acc():
            o_ref[...] = jnp.maximum(o_ref[...], col)


def _compact_kernel(supp_ref, data_ref, o_ref):
    supp = supp_ref[...].reshape(_NPAD // 128, 128)
    r_i = jax.lax.broadcasted_iota(jnp.int32, (40, 128), 0)
    l_i = jax.lax.broadcasted_iota(jnp.int32, (40, 128), 1)
    keep = (supp == 0.0) & ((r_i * 128 + l_i) < _N)
    kf = jnp.where(keep, 1.0, 0.0)

    u_r = jax.lax.broadcasted_iota(jnp.int32, (128, 128), 0)
    u_c = jax.lax.broadcasted_iota(jnp.int32, (128, 128), 1)
    upper = jnp.where(u_r <= u_c, 1.0, 0.0)
    incl = jnp.dot(kf, upper)

    l_r = jax.lax.broadcasted_iota(jnp.int32, (40, 40), 0)
    l_c = jax.lax.broadcasted_iota(jnp.int32, (40, 40), 1)
    lower = jnp.where(l_r > l_c, 1.0, 0.0)
    offs = jnp.dot(lower, incl[:, 127:128], precision=_HIGH)

    rank = (incl + offs - kf).astype(jnp.int32)
    slot = jnp.where(keep, rank, jnp.int32(2**30))
    slot_flat = slot.reshape(1, _NPAD)
    p_i = jax.lax.broadcasted_iota(jnp.int32, (_SLOTS, 1), 0)
    onehot = jnp.where(p_i == slot_flat, 1.0, 0.0)
    o_ref[...] = jnp.dot(onehot, data_ref[...], precision=_HIGH)


@functools.partial(jax.jit, static_argnames=("interpret",))
def kernel(boxes, scores, interpret=False):
    neg, x1c, y1c, x2c, y2c = jax.lax.sort(
        (-scores, boxes[:, 0], boxes[:, 1], boxes[:, 2], boxes[:, 3]),
        num_keys=1, is_stable=True)
    s = -neg
    b = jnp.stack([x1c, y1c, x2c, y2c], axis=1)
    pad = _NPAD - _N
    b = jnp.pad(b, ((0, pad), (0, 0)))
    s = jnp.pad(s, (0, pad), constant_values=-1.0)
    bt = b.T

    supp = pl.pallas_call(
        _supp_kernel,
        grid_spec=pltpu.PrefetchScalarGridSpec(
            num_scalar_prefetch=1,
            grid=(len(_STEPS),),
            in_specs=[
                pl.BlockSpec((_R, 4), lambda i, tb: (tb[1, i], 0)),
                pl.BlockSpec((4, _C), lambda i, tb: (0, tb[0, i])),
            ],
            out_specs=pl.BlockSpec((1, _C // 128, 128),
                                   lambda i, tb: (tb[0, i], 0, 0)),
        ),
        out_shape=jax.ShapeDtypeStruct(
            (_NPAD // _C, _C // 128, 128), jnp.float32),
        interpret=interpret,
    )(jnp.asarray(_TABLE), b, bt)

    data = jnp.concatenate(
        [s[:, None], b, jnp.zeros((_NPAD, 3), jnp.float32)], axis=1)
    out8 = pl.pallas_call(
        _compact_kernel,
        out_shape=jax.ShapeDtypeStruct((_SLOTS, 8), jnp.float32),
        interpret=interpret,
    )(supp, data)

    return out8[:_K, :5]


# speedup vs baseline: 1.4752x; 1.1091x over previous
import functools

import jax
import jax.numpy as jnp
import numpy as np
from jax.experimental import pallas as pl
from jax.experimental.pallas import tpu as pltpu

_N = 5000
_K = 100
_NPAD = 5120
_R = 1024
_C = 1024
_SLOTS = 128
_HIGH = jax.lax.Precision.HIGHEST

_NB = _NPAD // _R
_STEPS = [(cb, rb) for cb in range(_NB) for rb in range(cb + 1)]
_TABLE = np.asarray(_STEPS, dtype=np.int32).T.copy()


def _supp_kernel(tb_ref, b_ref, bt_ref, o_ref):
    i = pl.program_id(0)
    cb = tb_ref[0, i]
    rb = tb_ref[1, i]

    b = b_ref[...]
    bt = bt_ref[...]
    x1i, y1i, x2i, y2i = b[:, 0:1], b[:, 1:2], b[:, 2:3], b[:, 3:4]
    x1j, y1j, x2j, y2j = bt[0:1, :], bt[1:2, :], bt[2:3, :], bt[3:4, :]

    iw = jnp.minimum(x2i, x2j) - jnp.maximum(x1i, x1j)
    ih = jnp.minimum(y2i, y2j) - jnp.maximum(y1i, y1j)
    inter = jnp.maximum(iw, 0.0) * jnp.maximum(ih, 0.0)
    ai = (x2i - x1i) * (y2i - y1i)
    aj = (x2j - x1j) * (y2j - y1j)
    union = (ai + aj) - inter
    ovl = inter > 0.5 * union

    @pl.when(rb == cb)
    def _diag():
        gi = jax.lax.broadcasted_iota(jnp.int32, (_R, 1), 0)
        gj = jax.lax.broadcasted_iota(jnp.int32, (1, _C), 1)
        col = jnp.where(jnp.any(ovl & (gi < gj), axis=0, keepdims=True),
                        1.0, 0.0).reshape(1, _C // 128, 128)
        @pl.when(rb == 0)
        def _init():
            o_ref[...] = col
        @pl.when(rb != 0)
        def _acc():
            o_ref[...] = jnp.maximum(o_ref[...], col)

    @pl.when(rb < cb)
    def _off():
        col = jnp.where(jnp.any(ovl, axis=0, keepdims=True),
                        1.0, 0.0).reshape(1, _C // 128, 128)
        @pl.when(rb == 0)
        def _init():
            o_ref[...] = col
        @pl.when(rb != 0)
        def _acc():
            o_ref[...] = jnp.maximum(o_ref[...], col)


def _compact_kernel(supp_ref, data_ref, o_ref):
    supp = supp_ref[...].reshape(_NPAD // 128, 128)
    r_i = jax.lax.broadcasted_iota(jnp.int32, (40, 128), 0)
    l_i = jax.lax.broadcasted_iota(jnp.int32, (40, 128), 1)
    keep = (supp == 0.0) & ((r_i * 128 + l_i) < _N)
    kf = jnp.where(keep, 1.0, 0.0)

    u_r = jax.lax.broadcasted_iota(jnp.int32, (128, 128), 0)
    u_c = jax.lax.broadcasted_iota(jnp.int32, (128, 128), 1)
    upper = jnp.where(u_r <= u_c, 1.0, 0.0)
    incl = jnp.dot(kf, upper)

    l_r = jax.lax.broadcasted_iota(jnp.int32, (40, 40), 0)
    l_c = jax.lax.broadcasted_iota(jnp.int32, (40, 40), 1)
    lower = jnp.where(l_r > l_c, 1.0, 0.0)
    offs = jnp.dot(lower, incl[:, 127:128], precision=_HIGH)

    rank = (incl + offs - kf).astype(jnp.int32)
    slot = jnp.where(keep, rank, jnp.int32(2**30))
    slot_flat = slot.reshape(1, _NPAD)
    p_i = jax.lax.broadcasted_iota(jnp.int32, (_SLOTS, 1), 0)
    onehot = jnp.where(p_i == slot_flat, 1.0, 0.0)
    o_ref[...] = jnp.dot(onehot, data_ref[...], precision=_HIGH)


@functools.partial(jax.jit, static_argnames=("interpret",))
def kernel(boxes, scores, interpret=False):
    neg, x1c, y1c, x2c, y2c = jax.lax.sort(
        (-scores, boxes[:, 0], boxes[:, 1], boxes[:, 2], boxes[:, 3]),
        num_keys=1, is_stable=True)
    s = -neg
    b = jnp.stack([x1c, y1c, x2c, y2c], axis=1)
    pad = _NPAD - _N
    b = jnp.pad(b, ((0, pad), (0, 0)))
    s = jnp.pad(s, (0, pad), constant_values=-1.0)
    bt = b.T

    supp = pl.pallas_call(
        _supp_kernel,
        grid_spec=pltpu.PrefetchScalarGridSpec(
            num_scalar_prefetch=1,
            grid=(len(_STEPS),),
            in_specs=[
                pl.BlockSpec((_R, 4), lambda i, tb: (tb[1, i], 0)),
                pl.BlockSpec((4, _C), lambda i, tb: (0, tb[0, i])),
            ],
            out_specs=pl.BlockSpec((1, _C // 128, 128),
                                   lambda i, tb: (tb[0, i], 0, 0)),
        ),
        out_shape=jax.ShapeDtypeStruct(
            (_NPAD // _C, _C // 128, 128), jnp.float32),
        interpret=interpret,
    )(jnp.asarray(_TABLE), b, bt)

    data = jnp.concatenate(
        [s[:, None], b, jnp.zeros((_NPAD, 3), jnp.float32)], axis=1)
    out8 = pl.pallas_call(
        _compact_kernel,
        out_shape=jax.ShapeDtypeStruct((_SLOTS, 8), jnp.float32),
        interpret=interpret,
    )(supp, data)

    return out8[:_K, :5]


# speedup vs baseline: 1.5109x; 1.0242x over previous
import functools

import jax
import jax.numpy as jnp
import numpy as np
from jax.experimental import pallas as pl
from jax.experimental.pallas import tpu as pltpu

_N = 5000
_K = 100
_NPAD = 5120
_R = 1280
_C = 1280
_SLOTS = 128
_HIGH = jax.lax.Precision.HIGHEST

_NB = _NPAD // _R
_STEPS = [(cb, rb) for cb in range(_NB) for rb in range(cb + 1)]
_TABLE = np.asarray(_STEPS, dtype=np.int32).T.copy()


def _supp_kernel(tb_ref, b_ref, bt_ref, o_ref):
    i = pl.program_id(0)
    cb = tb_ref[0, i]
    rb = tb_ref[1, i]

    b = b_ref[...]
    bt = bt_ref[...]
    x1i, y1i, x2i, y2i = b[:, 0:1], b[:, 1:2], b[:, 2:3], b[:, 3:4]
    x1j, y1j, x2j, y2j = bt[0:1, :], bt[1:2, :], bt[2:3, :], bt[3:4, :]

    iw = jnp.minimum(x2i, x2j) - jnp.maximum(x1i, x1j)
    ih = jnp.minimum(y2i, y2j) - jnp.maximum(y1i, y1j)
    inter = jnp.maximum(iw, 0.0) * jnp.maximum(ih, 0.0)
    ai = (x2i - x1i) * (y2i - y1i)
    aj = (x2j - x1j) * (y2j - y1j)
    union = (ai + aj) - inter
    ovl = inter > 0.5 * union

    @pl.when(rb == cb)
    def _diag():
        gi = jax.lax.broadcasted_iota(jnp.int32, (_R, 1), 0)
        gj = jax.lax.broadcasted_iota(jnp.int32, (1, _C), 1)
        col = jnp.where(jnp.any(ovl & (gi < gj), axis=0, keepdims=True),
                        1.0, 0.0).reshape(1, _C // 128, 128)
        @pl.when(rb == 0)
        def _init():
            o_ref[...] = col
        @pl.when(rb != 0)
        def _acc():
            o_ref[...] = jnp.maximum(o_ref[...], col)

    @pl.when(rb < cb)
    def _off():
        col = jnp.where(jnp.any(ovl, axis=0, keepdims=True),
                        1.0, 0.0).reshape(1, _C // 128, 128)
        @pl.when(rb == 0)
        def _init():
            o_ref[...] = col
        @pl.when(rb != 0)
        def _acc():
            o_ref[...] = jnp.maximum(o_ref[...], col)


def _compact_kernel(supp_ref, data_ref, o_ref):
    supp = supp_ref[...].reshape(_NPAD // 128, 128)
    r_i = jax.lax.broadcasted_iota(jnp.int32, (40, 128), 0)
    l_i = jax.lax.broadcasted_iota(jnp.int32, (40, 128), 1)
    keep = (supp == 0.0) & ((r_i * 128 + l_i) < _N)
    kf = jnp.where(keep, 1.0, 0.0)

    u_r = jax.lax.broadcasted_iota(jnp.int32, (128, 128), 0)
    u_c = jax.lax.broadcasted_iota(jnp.int32, (128, 128), 1)
    upper = jnp.where(u_r <= u_c, 1.0, 0.0)
    incl = jnp.dot(kf, upper)

    l_r = jax.lax.broadcasted_iota(jnp.int32, (40, 40), 0)
    l_c = jax.lax.broadcasted_iota(jnp.int32, (40, 40), 1)
    lower = jnp.where(l_r > l_c, 1.0, 0.0)
    offs = jnp.dot(lower, incl[:, 127:128], precision=_HIGH)

    rank = (incl + offs - kf).astype(jnp.int32)
    slot = jnp.where(keep, rank, jnp.int32(2**30))
    slot_flat = slot.reshape(1, _NPAD)
    p_i = jax.lax.broadcasted_iota(jnp.int32, (_SLOTS, 1), 0)
    onehot = jnp.where(p_i == slot_flat, 1.0, 0.0)
    o_ref[...] = jnp.dot(onehot, data_ref[...], precision=_HIGH)


@functools.partial(jax.jit, static_argnames=("interpret",))
def kernel(boxes, scores, interpret=False):
    neg, x1c, y1c, x2c, y2c = jax.lax.sort(
        (-scores, boxes[:, 0], boxes[:, 1], boxes[:, 2], boxes[:, 3]),
        num_keys=1, is_stable=True)
    s = -neg
    b = jnp.stack([x1c, y1c, x2c, y2c], axis=1)
    pad = _NPAD - _N
    b = jnp.pad(b, ((0, pad), (0, 0)))
    s = jnp.pad(s, (0, pad), constant_values=-1.0)
    bt = b.T

    supp = pl.pallas_call(
        _supp_kernel,
        grid_spec=pltpu.PrefetchScalarGridSpec(
            num_scalar_prefetch=1,
            grid=(len(_STEPS),),
            in_specs=[
                pl.BlockSpec((_R, 4), lambda i, tb: (tb[1, i], 0)),
                pl.BlockSpec((4, _C), lambda i, tb: (0, tb[0, i])),
            ],
            out_specs=pl.BlockSpec((1, _C // 128, 128),
                                   lambda i, tb: (tb[0, i], 0, 0)),
        ),
        out_shape=jax.ShapeDtypeStruct(
            (_NPAD // _C, _C // 128, 128), jnp.float32),
        interpret=interpret,
    )(jnp.asarray(_TABLE), b, bt)

    data = jnp.concatenate(
        [s[:, None], b, jnp.zeros((_NPAD, 3), jnp.float32)], axis=1)
    out8 = pl.pallas_call(
        _compact_kernel,
        out_shape=jax.ShapeDtypeStruct((_SLOTS, 8), jnp.float32),
        interpret=interpret,
    )(supp, data)

    return out8[:_K, :5]


# speedup vs baseline: 1.5442x; 1.0220x over previous
import functools

import jax
import jax.numpy as jnp
import numpy as np
from jax.experimental import pallas as pl
from jax.experimental.pallas import tpu as pltpu

_N = 5000
_K = 100
_NPAD = 5120
_R = 1280
_C = 1280
_SLOTS = 128
_HIGH = jax.lax.Precision.HIGHEST

_NB = _NPAD // _R
_STEPS = [(cb, rb) for cb in range(_NB) for rb in range(cb + 1)]
_TABLE = np.asarray(_STEPS, dtype=np.int32).T.copy()


def _supp_kernel(tb_ref, b_ref, bt_ref, o_ref):
    i = pl.program_id(0)
    cb = tb_ref[0, i]
    rb = tb_ref[1, i]

    b = b_ref[...]
    bt = bt_ref[...]
    x1i, y1i, x2i, y2i = b[:, 0:1], b[:, 1:2], b[:, 2:3], b[:, 3:4]
    x1j, y1j, x2j, y2j = bt[0:1, :], bt[1:2, :], bt[2:3, :], bt[3:4, :]

    iw = jnp.minimum(x2i, x2j) - jnp.maximum(x1i, x1j)
    ih = jnp.minimum(y2i, y2j) - jnp.maximum(y1i, y1j)
    inter = jnp.maximum(iw, 0.0) * jnp.maximum(ih, 0.0)
    ai = (x2i - x1i) * (y2i - y1i)
    aj = (x2j - x1j) * (y2j - y1j)
    union = (ai + aj) - inter
    d = (inter + inter) - union

    @pl.when(rb == cb)
    def _diag():
        gi = jax.lax.broadcasted_iota(jnp.int32, (_R, 1), 0)
        gj = jax.lax.broadcasted_iota(jnp.int32, (1, _C), 1)
        col = jnp.max(jnp.where(gi < gj, d, -1.0), axis=0,
                      keepdims=True).reshape(1, _C // 128, 128)
        @pl.when(rb == 0)
        def _init():
            o_ref[...] = col
        @pl.when(rb != 0)
        def _acc():
            o_ref[...] = jnp.maximum(o_ref[...], col)

    @pl.when(rb < cb)
    def _off():
        col = jnp.max(d, axis=0, keepdims=True).reshape(1, _C // 128, 128)
        @pl.when(rb == 0)
        def _init():
            o_ref[...] = col
        @pl.when(rb != 0)
        def _acc():
            o_ref[...] = jnp.maximum(o_ref[...], col)


def _compact_kernel(supp_ref, data_ref, o_ref):
    supp = supp_ref[...].reshape(_NPAD // 128, 128)
    r_i = jax.lax.broadcasted_iota(jnp.int32, (40, 128), 0)
    l_i = jax.lax.broadcasted_iota(jnp.int32, (40, 128), 1)
    keep = (supp <= 0.0) & ((r_i * 128 + l_i) < _N)
    kf = jnp.where(keep, 1.0, 0.0)

    u_r = jax.lax.broadcasted_iota(jnp.int32, (128, 128), 0)
    u_c = jax.lax.broadcasted_iota(jnp.int32, (128, 128), 1)
    upper = jnp.where(u_r <= u_c, 1.0, 0.0)
    incl = jnp.dot(kf, upper)

    l_r = jax.lax.broadcasted_iota(jnp.int32, (40, 40), 0)
    l_c = jax.lax.broadcasted_iota(jnp.int32, (40, 40), 1)
    lower = jnp.where(l_r > l_c, 1.0, 0.0)
    offs = jnp.dot(lower, incl[:, 127:128], precision=_HIGH)

    rank = (incl + offs - kf).astype(jnp.int32)
    slot = jnp.where(keep, rank, jnp.int32(2**30))
    slot_flat = slot.reshape(1, _NPAD)
    p_i = jax.lax.broadcasted_iota(jnp.int32, (_SLOTS, 1), 0)
    onehot = jnp.where(p_i == slot_flat, 1.0, 0.0)
    o_ref[...] = jnp.dot(onehot, data_ref[...], precision=_HIGH)


@functools.partial(jax.jit, static_argnames=("interpret",))
def kernel(boxes, scores, interpret=False):
    neg, x1c, y1c, x2c, y2c = jax.lax.sort(
        (-scores, boxes[:, 0], boxes[:, 1], boxes[:, 2], boxes[:, 3]),
        num_keys=1, is_stable=True)
    s = -neg
    b = jnp.stack([x1c, y1c, x2c, y2c], axis=1)
    pad = _NPAD - _N
    b = jnp.pad(b, ((0, pad), (0, 0)))
    s = jnp.pad(s, (0, pad), constant_values=-1.0)
    bt = b.T

    supp = pl.pallas_call(
        _supp_kernel,
        grid_spec=pltpu.PrefetchScalarGridSpec(
            num_scalar_prefetch=1,
            grid=(len(_STEPS),),
            in_specs=[
                pl.BlockSpec((_R, 4), lambda i, tb: (tb[1, i], 0)),
                pl.BlockSpec((4, _C), lambda i, tb: (0, tb[0, i])),
            ],
            out_specs=pl.BlockSpec((1, _C // 128, 128),
                                   lambda i, tb: (tb[0, i], 0, 0)),
        ),
        out_shape=jax.ShapeDtypeStruct(
            (_NPAD // _C, _C // 128, 128), jnp.float32),
        interpret=interpret,
    )(jnp.asarray(_TABLE), b, bt)

    data = jnp.concatenate(
        [s[:, None], b, jnp.zeros((_NPAD, 3), jnp.float32)], axis=1)
    out8 = pl.pallas_call(
        _compact_kernel,
        out_shape=jax.ShapeDtypeStruct((_SLOTS, 8), jnp.float32),
        interpret=interpret,
    )(supp, data)

    return out8[:_K, :5]
